# Initial kernel scaffold; baseline (speedup 1.0000x reference)
import jax
import jax.numpy as jnp
from jax.experimental import pallas as pl


def kernel(x, adj, W, b):
    raise NotImplementedError("write your pallas kernel here")


# speedup vs baseline: 25.1732x; 25.1732x over previous
import functools

import jax
import jax.numpy as jnp
from jax import lax
from jax.experimental import pallas as pl
from jax.experimental.pallas import tpu as pltpu
from jax.experimental.pallas import tpu_sc as plsc

N = 10000
D = 128
E = 320000
NP = 10240
EP = 327680
PADN = EP - E
PADR = 240

C_SQ = 1.0
EPS = 4e-3
MIN_NORM = 1e-15

NC = 2
NS = 16
NW = NC * NS
EPW = EP // NW
CH = 128
CHA = 2048
RPS = NP // NS

_mesh = plsc.VectorSubcoreMesh(core_axis_name="c", subcore_axis_name="s")
_sc_params = pltpu.CompilerParams(needs_layout_passes=False)


@functools.partial(
    pl.kernel,
    mesh=_mesh,
    out_type=jax.ShapeDtypeStruct((NW * NP,), jnp.float32),
    scratch_types=[
        pltpu.VMEM((CHA,), jnp.int32),
        pltpu.VMEM((NP,), jnp.float32),
    ],
    compiler_params=_sc_params,
)
def _hist_kernel(dst_hbm, out_hbm, idx_v, hist_v):
    c = lax.axis_index("c")
    s = lax.axis_index("s")
    wid = c * NS + s
    zeros16 = jnp.zeros((16,), jnp.float32)
    ones16 = jnp.ones((16,), jnp.float32)

    def zero_body(i, _):
        hist_v[pl.ds(i * 16, 16)] = zeros16
        return 0

    lax.fori_loop(0, NP // 16, zero_body, 0)

    def chunk_body(k, _):
        base = wid * EPW + k * CHA
        pltpu.sync_copy(dst_hbm.at[pl.ds(base, CHA)], idx_v)

        def inner(j, _):
            idx = idx_v[pl.ds(j * 16, 16)]
            plsc.addupdate_scatter(hist_v, [idx], ones16)
            return 0

        lax.fori_loop(0, CHA // 16, inner, 0)
        return 0

    lax.fori_loop(0, EPW // CHA, chunk_body, 0)
    pltpu.sync_copy(hist_v, out_hbm.at[pl.ds(wid * NP, NP)])


@functools.partial(
    pl.kernel,
    mesh=_mesh,
    out_type=jax.ShapeDtypeStruct((NC * NP, D), jnp.float32),
    scratch_types=[
        pltpu.VMEM((CH,), jnp.int32),
        pltpu.VMEM((CH,), jnp.int32),
        pltpu.VMEM((CH, D), jnp.float32),
        pltpu.VMEM_SHARED((NP, D), jnp.float32),
        pltpu.SemaphoreType.DMA,
    ],
    compiler_params=_sc_params,
)
def _agg_kernel(src_hbm, dst_hbm, y_hbm, out_hbm, idx_s, idx_d, rows, acc_sh,
                sem):
    c = lax.axis_index("c")
    s = lax.axis_index("s")
    wid = c * NS + s
    zeros16 = jnp.zeros((16,), jnp.float32)

    def zero_rows(i, _):
        rows[i // 8, pl.ds((i % 8) * 16, 16)] = zeros16
        return 0

    lax.fori_loop(0, CH * D // 16, zero_rows, 0)

    def zero_acc(i, _):
        pltpu.sync_copy(rows, acc_sh.at[pl.ds(s * RPS + i * CH, CH)])
        return 0

    lax.fori_loop(0, RPS // CH, zero_acc, 0)
    plsc.subcore_barrier()

    def chunk_body(k, _):
        base = wid * EPW + k * CH
        pltpu.sync_copy(src_hbm.at[pl.ds(base, CH)], idx_s)
        pltpu.sync_copy(dst_hbm.at[pl.ds(base, CH)], idx_d)
        pltpu.async_copy(y_hbm.at[idx_s], rows, sem).wait()
        pltpu.sync_copy(rows, acc_sh.at[idx_d], add=True)
        return 0

    lax.fori_loop(0, EPW // CH, chunk_body, 0)
    plsc.subcore_barrier()

    def export_body(i, _):
        off = s * RPS + i * CH
        pltpu.sync_copy(acc_sh.at[pl.ds(off, CH)], rows)
        pltpu.sync_copy(rows, out_hbm.at[pl.ds(c * NP + off, CH)])
        return 0

    lax.fori_loop(0, RPS // CH, export_body, 0)


def _dense_body(x_ref, w_ref, hist_ref, y_ref):
    deg = jnp.sum(hist_ref[...], axis=0)[:N] + 1.0
    dinv = lax.rsqrt(deg)
    y = jnp.dot(x_ref[...], w_ref[...],
                preferred_element_type=jnp.float32) * dinv[:, None]
    y_ref[0:N, :] = y
    y_ref[N:NP, :] = jnp.zeros((NP - N, D), jnp.float32)


def _final_body(acc_ref, y_ref, hist_ref, b_ref, o_ref):
    deg = jnp.sum(hist_ref[...], axis=0)[:N] + 1.0
    dinv = lax.rsqrt(deg)
    agg = acc_ref[0:N, :] + acc_ref[NP:NP + N, :] + y_ref[0:N, :]
    s_t = agg * dinv[:, None] + b_ref[...][None, :]
    nrm = jnp.maximum(
        jnp.sqrt(jnp.sum(s_t * s_t, axis=1, keepdims=True)), MIN_NORM)
    e = jnp.tanh(nrm) * s_t / nrm
    n2 = jnp.maximum(
        jnp.sqrt(jnp.sum(e * e, axis=1, keepdims=True)), MIN_NORM)
    maxnorm = 1.0 - EPS
    o_ref[...] = jnp.where(n2 > maxnorm, e / n2 * maxnorm, e)


def kernel(x, adj, W, b):
    src = adj[0]
    dst = adj[1]
    pad = (N + (lax.iota(jnp.int32, PADN) % PADR)).astype(adj.dtype)
    src_e = jnp.concatenate([src, pad])
    dst_e = jnp.concatenate([dst, pad])

    hist_flat = _hist_kernel(dst_e)
    hist = hist_flat.reshape(NW, NP)

    y = pl.pallas_call(
        _dense_body,
        out_shape=jax.ShapeDtypeStruct((NP, D), jnp.float32),
    )(x, W, hist)

    acc = _agg_kernel(src_e, dst_e, y)

    out = pl.pallas_call(
        _final_body,
        out_shape=jax.ShapeDtypeStruct((N, D), jnp.float32),
    )(acc, y, hist, b)
    return out


# speedup vs baseline: 44.6916x; 1.7754x over previous
import functools

import jax
import jax.numpy as jnp
from jax import lax
from jax.experimental import pallas as pl
from jax.experimental.pallas import tpu as pltpu
from jax.experimental.pallas import tpu_sc as plsc

N = 10000
D = 128
E = 320000
NP = 10240
EP = 327680
PADN = EP - E
PADR = 240

C_SQ = 1.0
EPS = 4e-3
MIN_NORM = 1e-15

NC = 2
NS = 16
NW = NC * NS
EPW = EP // NW
CH = 128
CHA = 2048
RPS = NP // NS

_mesh = plsc.VectorSubcoreMesh(core_axis_name="c", subcore_axis_name="s")
_sc_params = pltpu.CompilerParams(needs_layout_passes=False)


@functools.partial(
    pl.kernel,
    mesh=_mesh,
    out_type=jax.ShapeDtypeStruct((NW * NP,), jnp.float32),
    scratch_types=[
        pltpu.VMEM((CHA,), jnp.int32),
        pltpu.VMEM((NP,), jnp.float32),
    ],
    compiler_params=_sc_params,
)
def _hist_kernel(dst_hbm, out_hbm, idx_v, hist_v):
    c = lax.axis_index("c")
    s = lax.axis_index("s")
    wid = c * NS + s
    zeros16 = jnp.zeros((16,), jnp.float32)
    ones16 = jnp.ones((16,), jnp.float32)

    def zero_body(i, _):
        hist_v[pl.ds(i * 16, 16)] = zeros16
        return 0

    lax.fori_loop(0, NP // 16, zero_body, 0)

    def chunk_body(k, _):
        base = wid * EPW + k * CHA
        pltpu.sync_copy(dst_hbm.at[pl.ds(base, CHA)], idx_v)

        def inner(j, _):
            idx = idx_v[pl.ds(j * 16, 16)]
            plsc.addupdate_scatter(hist_v, [idx], ones16)
            return 0

        lax.fori_loop(0, CHA // 16, inner, 0)
        return 0

    lax.fori_loop(0, EPW // CHA, chunk_body, 0)
    pltpu.sync_copy(hist_v, out_hbm.at[pl.ds(wid * NP, NP)])


NCH = EPW // CH
NHALF = 2
HCH = NCH // NHALF
NBUF = 2


@functools.partial(
    pl.kernel,
    mesh=_mesh,
    out_type=jax.ShapeDtypeStruct((NC * NP, D), jnp.float32),
    scratch_types=[
        pltpu.VMEM((HCH, CH), jnp.int32),
        pltpu.VMEM((HCH, CH), jnp.int32),
        pltpu.VMEM((NBUF, CH, D), jnp.float32),
        pltpu.VMEM_SHARED((NP, D), jnp.float32),
        [pltpu.SemaphoreType.DMA] * NBUF,
        [pltpu.SemaphoreType.DMA] * NBUF,
    ],
    compiler_params=_sc_params,
)
def _agg_kernel(src_hbm, dst_hbm, y_hbm, out_hbm, idx_s, idx_d, rows, acc_sh,
                gsem, ssem):
    c = lax.axis_index("c")
    s = lax.axis_index("s")
    wid = c * NS + s
    zeros16 = jnp.zeros((16,), jnp.float32)

    def zero_rows(i, _):
        rows[0, i // 8, pl.ds((i % 8) * 16, 16)] = zeros16
        return 0

    lax.fori_loop(0, CH * D // 16, zero_rows, 0)

    def zero_acc(i, _):
        pltpu.sync_copy(rows.at[0], acc_sh.at[pl.ds(s * RPS + i * CH, CH)])
        return 0

    lax.fori_loop(0, RPS // CH, zero_acc, 0)
    plsc.subcore_barrier()

    for h in range(NHALF):
        pltpu.sync_copy(src_hbm.at[wid, pl.ds(h * HCH, HCH)], idx_s)
        pltpu.sync_copy(dst_hbm.at[wid, pl.ds(h * HCH, HCH)], idx_d)

        for b in range(NBUF):
            pltpu.async_copy(y_hbm.at[idx_s.at[b]], rows.at[b], gsem[b])

        def ring_body(kk, _):
            for b in range(NBUF):
                k = kk * NBUF + b
                pltpu.make_async_copy(y_hbm.at[idx_s.at[b]], rows.at[b],
                                      gsem[b]).wait()
                pltpu.async_copy(rows.at[b], acc_sh.at[idx_d.at[k]], ssem[b],
                                 add=True)
                pltpu.make_async_copy(rows.at[b], acc_sh.at[idx_d.at[b]],
                                      ssem[b]).wait()

                @pl.when(kk < HCH // NBUF - 1)
                def _():
                    pltpu.async_copy(y_hbm.at[idx_s.at[k + NBUF]], rows.at[b],
                                     gsem[b])
            return 0

        lax.fori_loop(0, HCH // NBUF, ring_body, 0)
    plsc.subcore_barrier()

    def export_body(i, _):
        off = s * RPS + i * CH
        pltpu.sync_copy(acc_sh.at[pl.ds(off, CH)], rows.at[0])
        pltpu.sync_copy(rows.at[0], out_hbm.at[pl.ds(c * NP + off, CH)])
        return 0

    lax.fori_loop(0, RPS // CH, export_body, 0)


def _dense_body(x_ref, w_ref, hist_ref, y_ref):
    deg = jnp.sum(hist_ref[...], axis=0)[:N] + 1.0
    dinv = lax.rsqrt(deg)
    y = jnp.dot(x_ref[...], w_ref[...],
                preferred_element_type=jnp.float32) * dinv[:, None]
    y_ref[0:N, :] = y
    y_ref[N:NP, :] = jnp.zeros((NP - N, D), jnp.float32)


def _final_body(acc_ref, y_ref, hist_ref, b_ref, o_ref):
    deg = jnp.sum(hist_ref[...], axis=0)[:N] + 1.0
    dinv = lax.rsqrt(deg)
    agg = acc_ref[0:N, :] + acc_ref[NP:NP + N, :] + y_ref[0:N, :]
    s_t = agg * dinv[:, None] + b_ref[...][None, :]
    nrm = jnp.maximum(
        jnp.sqrt(jnp.sum(s_t * s_t, axis=1, keepdims=True)), MIN_NORM)
    e = jnp.tanh(nrm) * s_t / nrm
    n2 = jnp.maximum(
        jnp.sqrt(jnp.sum(e * e, axis=1, keepdims=True)), MIN_NORM)
    maxnorm = 1.0 - EPS
    o_ref[...] = jnp.where(n2 > maxnorm, e / n2 * maxnorm, e)


def kernel(x, adj, W, b):
    src = adj[0]
    dst = adj[1]
    pad = (N + (lax.iota(jnp.int32, PADN) % PADR)).astype(adj.dtype)
    src_e = jnp.concatenate([src, pad])
    dst_e = jnp.concatenate([dst, pad])

    src3 = src_e.reshape(NW, NCH, CH)
    dst3 = dst_e.reshape(NW, NCH, CH)

    hist_flat = _hist_kernel(dst_e)
    hist = hist_flat.reshape(NW, NP)

    y = pl.pallas_call(
        _dense_body,
        out_shape=jax.ShapeDtypeStruct((NP, D), jnp.float32),
    )(x, W, hist)

    acc = _agg_kernel(src3, dst3, y)

    out = pl.pallas_call(
        _final_body,
        out_shape=jax.ShapeDtypeStruct((N, D), jnp.float32),
    )(acc, y, hist, b)
    return out


# speedup vs baseline: 46.2241x; 1.0343x over previous
import functools

import jax
import jax.numpy as jnp
from jax import lax
from jax.experimental import pallas as pl
from jax.experimental.pallas import tpu as pltpu
from jax.experimental.pallas import tpu_sc as plsc

N = 10000
D = 128
E = 320000
NP = 10240
EP = 327680
PADN = EP - E
PADR = 240

C_SQ = 1.0
EPS = 4e-3
MIN_NORM = 1e-15

NC = 2
NS = 16
NW = NC * NS
EPW = EP // NW
CH = 64
CHA = 2048
RPS = NP // NS

_mesh = plsc.VectorSubcoreMesh(core_axis_name="c", subcore_axis_name="s")
_sc_params = pltpu.CompilerParams(needs_layout_passes=False)


@functools.partial(
    pl.kernel,
    mesh=_mesh,
    out_type=jax.ShapeDtypeStruct((NW * NP,), jnp.float32),
    scratch_types=[
        pltpu.VMEM((CHA,), jnp.int32),
        pltpu.VMEM((NP,), jnp.float32),
    ],
    compiler_params=_sc_params,
)
def _hist_kernel(dst_hbm, out_hbm, idx_v, hist_v):
    c = lax.axis_index("c")
    s = lax.axis_index("s")
    wid = c * NS + s
    zeros16 = jnp.zeros((16,), jnp.float32)
    ones16 = jnp.ones((16,), jnp.float32)

    def zero_body(i, _):
        hist_v[pl.ds(i * 16, 16)] = zeros16
        return 0

    lax.fori_loop(0, NP // 16, zero_body, 0)

    def chunk_body(k, _):
        base = wid * EPW + k * CHA
        pltpu.sync_copy(dst_hbm.at[pl.ds(base, CHA)], idx_v)

        def inner(j, _):
            idx = idx_v[pl.ds(j * 16, 16)]
            plsc.addupdate_scatter(hist_v, [idx], ones16)
            return 0

        lax.fori_loop(0, CHA // 16, inner, 0)
        return 0

    lax.fori_loop(0, EPW // CHA, chunk_body, 0)
    pltpu.sync_copy(hist_v, out_hbm.at[pl.ds(wid * NP, NP)])


NCH = EPW // CH
NHALF = 4
HCH = NCH // NHALF
NBUF = 4


@functools.partial(
    pl.kernel,
    mesh=_mesh,
    out_type=jax.ShapeDtypeStruct((NC * NP, D), jnp.float32),
    scratch_types=[
        pltpu.VMEM((HCH, CH), jnp.int32),
        pltpu.VMEM((HCH, CH), jnp.int32),
        pltpu.VMEM((NBUF, CH, D), jnp.float32),
        pltpu.VMEM_SHARED((NP, D), jnp.float32),
        [pltpu.SemaphoreType.DMA] * NBUF,
        [pltpu.SemaphoreType.DMA] * NBUF,
    ],
    compiler_params=_sc_params,
)
def _agg_kernel(src_hbm, dst_hbm, y_hbm, out_hbm, idx_s, idx_d, rows, acc_sh,
                gsem, ssem):
    c = lax.axis_index("c")
    s = lax.axis_index("s")
    wid = c * NS + s
    zeros16 = jnp.zeros((16,), jnp.float32)

    def zero_rows(i, _):
        rows[0, i // 8, pl.ds((i % 8) * 16, 16)] = zeros16
        return 0

    lax.fori_loop(0, CH * D // 16, zero_rows, 0)

    def zero_acc(i, _):
        pltpu.sync_copy(rows.at[0], acc_sh.at[pl.ds(s * RPS + i * CH, CH)])
        return 0

    lax.fori_loop(0, RPS // CH, zero_acc, 0)
    plsc.subcore_barrier()

    for h in range(NHALF):
        pltpu.sync_copy(src_hbm.at[wid, pl.ds(h * HCH, HCH)], idx_s)
        pltpu.sync_copy(dst_hbm.at[wid, pl.ds(h * HCH, HCH)], idx_d)

        for b in range(NBUF):
            pltpu.async_copy(y_hbm.at[idx_s.at[b]], rows.at[b], gsem[b])

        def ring_body(kk, _):
            for b in range(NBUF):
                k = kk * NBUF + b
                pltpu.make_async_copy(y_hbm.at[idx_s.at[b]], rows.at[b],
                                      gsem[b]).wait()
                pltpu.async_copy(rows.at[b], acc_sh.at[idx_d.at[k]], ssem[b],
                                 add=True)
                pltpu.make_async_copy(rows.at[b], acc_sh.at[idx_d.at[b]],
                                      ssem[b]).wait()

                @pl.when(kk < HCH // NBUF - 1)
                def _():
                    pltpu.async_copy(y_hbm.at[idx_s.at[k + NBUF]], rows.at[b],
                                     gsem[b])
            return 0

        lax.fori_loop(0, HCH // NBUF, ring_body, 0)
    plsc.subcore_barrier()

    def export_body(i, _):
        off = s * RPS + i * CH
        pltpu.sync_copy(acc_sh.at[pl.ds(off, CH)], rows.at[0])
        pltpu.sync_copy(rows.at[0], out_hbm.at[pl.ds(c * NP + off, CH)])
        return 0

    lax.fori_loop(0, RPS // CH, export_body, 0)


def _dense_body(x_ref, w_ref, hist_ref, y_ref):
    deg = jnp.sum(hist_ref[...], axis=0)[:N] + 1.0
    dinv = lax.rsqrt(deg)
    y = jnp.dot(x_ref[...], w_ref[...],
                preferred_element_type=jnp.float32) * dinv[:, None]
    y_ref[0:N, :] = y
    y_ref[N:NP, :] = jnp.zeros((NP - N, D), jnp.float32)


def _final_body(acc_ref, y_ref, hist_ref, b_ref, o_ref):
    deg = jnp.sum(hist_ref[...], axis=0)[:N] + 1.0
    dinv = lax.rsqrt(deg)
    agg = acc_ref[0:N, :] + acc_ref[NP:NP + N, :] + y_ref[0:N, :]
    s_t = agg * dinv[:, None] + b_ref[...][None, :]
    nrm = jnp.maximum(
        jnp.sqrt(jnp.sum(s_t * s_t, axis=1, keepdims=True)), MIN_NORM)
    e = jnp.tanh(nrm) * s_t / nrm
    n2 = jnp.maximum(
        jnp.sqrt(jnp.sum(e * e, axis=1, keepdims=True)), MIN_NORM)
    maxnorm = 1.0 - EPS
    o_ref[...] = jnp.where(n2 > maxnorm, e / n2 * maxnorm, e)


def kernel(x, adj, W, b):
    src = adj[0]
    dst = adj[1]
    pad = (N + (lax.iota(jnp.int32, PADN) % PADR)).astype(adj.dtype)
    src_e = jnp.concatenate([src, pad])
    dst_e = jnp.concatenate([dst, pad])

    src3 = src_e.reshape(NW, NCH, CH)
    dst3 = dst_e.reshape(NW, NCH, CH)

    hist_flat = _hist_kernel(dst_e)
    hist = hist_flat.reshape(NW, NP)

    y = pl.pallas_call(
        _dense_body,
        out_shape=jax.ShapeDtypeStruct((NP, D), jnp.float32),
    )(x, W, hist)

    acc = _agg_kernel(src3, dst3, y)

    out = pl.pallas_call(
        _final_body,
        out_shape=jax.ShapeDtypeStruct((N, D), jnp.float32),
    )(acc, y, hist, b)
    return out
